# baseline (speedup 1.0000x reference)
import functools

import jax
import jax.numpy as jnp
from jax import lax
from jax.experimental import pallas as pl
from jax.experimental.pallas import tpu as pltpu

_EPS = float(jnp.finfo(jnp.float32).eps)


def _pp_tp_body(nsteps, yt_ref, yp_ref, out_ref, pp_ref, tp_ref):
    i = pl.program_id(0)
    R, C = yt_ref.shape

    @pl.when(i == 0)
    def _init():
        pp_ref[...] = jnp.zeros_like(pp_ref)
        tp_ref[...] = jnp.zeros_like(tp_ref)

    xt = yt_ref[...]
    xp = yp_ref[...]
    iota = lax.broadcasted_iota(jnp.int32, (R, C), 1)
    big = jnp.int32(C + 1)
    mt = jnp.max(xt, axis=1, keepdims=True)
    mp = jnp.max(xp, axis=1, keepdims=True)
    t = jnp.min(jnp.where(xt == mt, iota, big), axis=1, keepdims=True)
    p = jnp.min(jnp.where(xp == mp, iota, big), axis=1, keepdims=True)
    onehot = (iota == p).astype(jnp.float32)
    eqf = (t == p).astype(jnp.float32)
    pp_ref[...] += jnp.sum(onehot, axis=0, keepdims=True)
    tp_ref[...] += jnp.sum(onehot * eqf, axis=0, keepdims=True)

    @pl.when(i == nsteps - 1)
    def _fin():
        pp = pp_ref[...]
        tp = tp_ref[...]
        prec = tp / (pp + _EPS)
        out_ref[...] = jnp.sum(prec, axis=1, keepdims=True) / jnp.float32(C)


def kernel(y_true, y_pred):
    N, C = y_true.shape
    R = 4000 if N % 4000 == 0 else N
    G = N // R
    out = pl.pallas_call(
        functools.partial(_pp_tp_body, G),
        grid=(G,),
        in_specs=[
            pl.BlockSpec((R, C), lambda i: (i, 0)),
            pl.BlockSpec((R, C), lambda i: (i, 0)),
        ],
        out_specs=pl.BlockSpec((1, 1), lambda i: (0, 0)),
        out_shape=jax.ShapeDtypeStruct((1, 1), jnp.float32),
        scratch_shapes=[
            pltpu.VMEM((1, C), jnp.float32),
            pltpu.VMEM((1, C), jnp.float32),
        ],
        compiler_params=pltpu.CompilerParams(dimension_semantics=("arbitrary",)),
    )(y_true, y_pred)
    return out[0, 0]


# speedup vs baseline: 1.3287x; 1.3287x over previous
import functools

import jax
import jax.numpy as jnp
from jax import lax
from jax.experimental import pallas as pl
from jax.experimental.pallas import tpu as pltpu

_EPS = float(jnp.finfo(jnp.float32).eps)


def _pp_tp_body(nsteps, yt_ref, yp_ref, out_ref, pp_ref, tp_ref):
    i = pl.program_id(0)
    R, C = yt_ref.shape

    @pl.when(i == 0)
    def _init():
        pp_ref[...] = jnp.zeros_like(pp_ref)
        tp_ref[...] = jnp.zeros_like(tp_ref)

    xt = yt_ref[...]
    xp = yp_ref[...]
    mt = jnp.max(xt, axis=1, keepdims=True)
    mp = jnp.max(xp, axis=1, keepdims=True)
    eq_t = xt == mt
    eq_p = xp == mp
    pp_ref[...] += jnp.sum(eq_p.astype(jnp.float32), axis=0, keepdims=True)
    tp_ref[...] += jnp.sum((eq_t & eq_p).astype(jnp.float32), axis=0, keepdims=True)

    @pl.when(i == nsteps - 1)
    def _fin():
        pp = pp_ref[...]
        tp = tp_ref[...]
        prec = tp / (pp + _EPS)
        out_ref[...] = jnp.sum(prec, axis=1, keepdims=True) / jnp.float32(C)


def kernel(y_true, y_pred):
    N, C = y_true.shape
    R = 4000 if N % 4000 == 0 else N
    G = N // R
    out = pl.pallas_call(
        functools.partial(_pp_tp_body, G),
        grid=(G,),
        in_specs=[
            pl.BlockSpec((R, C), lambda i: (i, 0)),
            pl.BlockSpec((R, C), lambda i: (i, 0)),
        ],
        out_specs=pl.BlockSpec((1, 1), lambda i: (0, 0)),
        out_shape=jax.ShapeDtypeStruct((1, 1), jnp.float32),
        scratch_shapes=[
            pltpu.VMEM((1, C), jnp.float32),
            pltpu.VMEM((1, C), jnp.float32),
        ],
        compiler_params=pltpu.CompilerParams(dimension_semantics=("arbitrary",)),
    )(y_true, y_pred)
    return out[0, 0]


# speedup vs baseline: 1.4537x; 1.0940x over previous
import functools

import jax
import jax.numpy as jnp
from jax import lax
from jax.experimental import pallas as pl
from jax.experimental.pallas import tpu as pltpu

_EPS = float(jnp.finfo(jnp.float32).eps)


def _pp_tp_body(nsteps, yt_ref, yp_ref, out_ref, pp_ref, tp_ref):
    i = pl.program_id(0)
    R, C = yt_ref.shape

    @pl.when(i == 0)
    def _init():
        pp_ref[...] = jnp.zeros_like(pp_ref)
        tp_ref[...] = jnp.zeros_like(tp_ref)

    xt = yt_ref[...]
    xp = yp_ref[...]
    mt = jnp.max(xt, axis=1, keepdims=True)
    mp = jnp.max(xp, axis=1, keepdims=True)
    eq_t = xt == mt
    eq_p = xp == mp
    pp_ref[...] += jnp.sum(eq_p.astype(jnp.float32), axis=0, keepdims=True)
    tp_ref[...] += jnp.sum((eq_t & eq_p).astype(jnp.float32), axis=0, keepdims=True)

    @pl.when(i == nsteps - 1)
    def _fin():
        pp = pp_ref[...]
        tp = tp_ref[...]
        prec = tp / (pp + _EPS)
        out_ref[...] = jnp.sum(prec, axis=1, keepdims=True) / jnp.float32(C)


def kernel(y_true, y_pred):
    N, C = y_true.shape
    R = 20000 if N % 20000 == 0 else N
    G = N // R
    out = pl.pallas_call(
        functools.partial(_pp_tp_body, G),
        grid=(G,),
        in_specs=[
            pl.BlockSpec((R, C), lambda i: (i, 0)),
            pl.BlockSpec((R, C), lambda i: (i, 0)),
        ],
        out_specs=pl.BlockSpec((1, 1), lambda i: (0, 0)),
        out_shape=jax.ShapeDtypeStruct((1, 1), jnp.float32),
        scratch_shapes=[
            pltpu.VMEM((1, C), jnp.float32),
            pltpu.VMEM((1, C), jnp.float32),
        ],
        compiler_params=pltpu.CompilerParams(dimension_semantics=("arbitrary",)),
    )(y_true, y_pred)
    return out[0, 0]


# speedup vs baseline: 1.4579x; 1.0029x over previous
import functools

import jax
import jax.numpy as jnp
from jax import lax
from jax.experimental import pallas as pl
from jax.experimental.pallas import tpu as pltpu

_EPS = float(jnp.finfo(jnp.float32).eps)


def _pp_tp_body(nsteps, yta_ref, ytb_ref, ypa_ref, ypb_ref, out_ref,
                pp_ref, tp_ref):
    i = pl.program_id(0)

    @pl.when(i == 0)
    def _init():
        pp_ref[...] = jnp.zeros_like(pp_ref)
        tp_ref[...] = jnp.zeros_like(tp_ref)

    for yt_ref, yp_ref in ((yta_ref, ypa_ref), (ytb_ref, ypb_ref)):
        xt = yt_ref[...]
        xp = yp_ref[...]
        mt = jnp.max(xt, axis=1, keepdims=True)
        mp = jnp.max(xp, axis=1, keepdims=True)
        eq_t = xt == mt
        eq_p = xp == mp
        pp_ref[...] += jnp.sum(eq_p.astype(jnp.float32), axis=0, keepdims=True)
        tp_ref[...] += jnp.sum((eq_t & eq_p).astype(jnp.float32), axis=0,
                               keepdims=True)

    @pl.when(i == nsteps - 1)
    def _fin():
        C = pp_ref.shape[1]
        prec = tp_ref[...] / (pp_ref[...] + _EPS)
        out_ref[...] = jnp.sum(prec, axis=1, keepdims=True) / jnp.float32(C)


def kernel(y_true, y_pred):
    N, C = y_true.shape
    R = 10000 if N % 20000 == 0 else N
    G = N // (2 * R) if N % 20000 == 0 else 1
    out = pl.pallas_call(
        functools.partial(_pp_tp_body, G),
        grid=(G,),
        in_specs=[
            pl.BlockSpec((R, C), lambda i: (i, 0)),
            pl.BlockSpec((R, C), lambda i, _G=G: (i + _G, 0)),
            pl.BlockSpec((R, C), lambda i: (i, 0)),
            pl.BlockSpec((R, C), lambda i, _G=G: (i + _G, 0)),
        ],
        out_specs=pl.BlockSpec((1, 1), lambda i: (0, 0)),
        out_shape=jax.ShapeDtypeStruct((1, 1), jnp.float32),
        scratch_shapes=[
            pltpu.VMEM((1, C), jnp.float32),
            pltpu.VMEM((1, C), jnp.float32),
        ],
        compiler_params=pltpu.CompilerParams(dimension_semantics=("arbitrary",)),
    )(y_true, y_true, y_pred, y_pred)
    return out[0, 0]


# speedup vs baseline: 5.6675x; 3.8875x over previous
import functools

import jax
import jax.numpy as jnp
from jax import lax
from jax.experimental import pallas as pl
from jax.experimental.pallas import tpu as pltpu

_EPS = float(jnp.finfo(jnp.float32).eps)


def _cls_body(nsteps, ytT_ref, ypT_ref, pp_out, tp_out, ppa_ref, tpa_ref):
    i = pl.program_id(0)

    @pl.when(i == 0)
    def _init():
        ppa_ref[...] = jnp.zeros_like(ppa_ref)
        tpa_ref[...] = jnp.zeros_like(tpa_ref)

    xt = ytT_ref[...]
    xp = ypT_ref[...]
    mt = jnp.max(xt, axis=0, keepdims=True)
    mp = jnp.max(xp, axis=0, keepdims=True)
    eq_t = xt == mt
    eq_p = xp == mp
    ppf = eq_p.astype(jnp.float32)
    tpf = (eq_t & eq_p).astype(jnp.float32)
    C, RC = xt.shape
    g = RC // 128
    accp = ppf[:, 0:128]
    acct = tpf[:, 0:128]
    for j in range(1, g):
        accp = accp + ppf[:, j * 128:(j + 1) * 128]
        acct = acct + tpf[:, j * 128:(j + 1) * 128]
    ppa_ref[...] += accp
    tpa_ref[...] += acct

    @pl.when(i == nsteps - 1)
    def _fin():
        pp_out[...] = jnp.sum(ppa_ref[...], axis=1, keepdims=True)
        tp_out[...] = jnp.sum(tpa_ref[...], axis=1, keepdims=True)


def _fin_body(ytr_ref, ypr_ref, ppm_ref, tpm_ref, out_ref):
    xt = ytr_ref[...]
    xp = ypr_ref[...]
    mt = jnp.max(xt, axis=0, keepdims=True)
    mp = jnp.max(xp, axis=0, keepdims=True)
    eq_t = xt == mt
    eq_p = xp == mp
    pp = ppm_ref[...] + jnp.sum(eq_p.astype(jnp.float32), axis=1, keepdims=True)
    tp = tpm_ref[...] + jnp.sum((eq_t & eq_p).astype(jnp.float32), axis=1,
                                keepdims=True)
    C = pp.shape[0]
    prec = tp / (pp + _EPS)
    out_ref[...] = jnp.sum(prec, axis=0, keepdims=True) / jnp.float32(C)


def kernel(y_true, y_pred):
    N, C = y_true.shape
    ytT = y_true.T
    ypT = y_pred.T
    RC = 11904
    G = N // RC
    nmain = G * RC
    pp_m, tp_m = pl.pallas_call(
        functools.partial(_cls_body, G),
        grid=(G,),
        in_specs=[
            pl.BlockSpec((C, RC), lambda i: (0, i)),
            pl.BlockSpec((C, RC), lambda i: (0, i)),
        ],
        out_specs=[
            pl.BlockSpec((C, 1), lambda i: (0, 0)),
            pl.BlockSpec((C, 1), lambda i: (0, 0)),
        ],
        out_shape=[
            jax.ShapeDtypeStruct((C, 1), jnp.float32),
            jax.ShapeDtypeStruct((C, 1), jnp.float32),
        ],
        scratch_shapes=[
            pltpu.VMEM((C, 128), jnp.float32),
            pltpu.VMEM((C, 128), jnp.float32),
        ],
        compiler_params=pltpu.CompilerParams(dimension_semantics=("arbitrary",)),
    )(ytT, ypT)

    ytR = lax.slice(ytT, (0, nmain), (C, N))
    ypR = lax.slice(ypT, (0, nmain), (C, N))
    out = pl.pallas_call(
        _fin_body,
        out_shape=jax.ShapeDtypeStruct((1, 1), jnp.float32),
    )(ytR, ypR, pp_m, tp_m)
    return out[0, 0]


# speedup vs baseline: 5.9215x; 1.0448x over previous
import functools

import jax
import jax.numpy as jnp
from jax import lax
from jax.experimental import pallas as pl
from jax.experimental.pallas import tpu as pltpu

_EPS = float(jnp.finfo(jnp.float32).eps)


def _cls_body(nsteps, ytT_ref, ypT_ref, pp_out, tp_out, ppa_ref, tpa_ref):
    i = pl.program_id(0)

    @pl.when(i == 0)
    def _init():
        ppa_ref[...] = jnp.zeros_like(ppa_ref)
        tpa_ref[...] = jnp.zeros_like(tpa_ref)

    xt = ytT_ref[...]
    xp = ypT_ref[...]
    mt = jnp.max(xt, axis=0, keepdims=True)
    mp = jnp.max(xp, axis=0, keepdims=True)
    eq_t = xt == mt
    eq_p = xp == mp
    ppf = eq_p.astype(jnp.float32)
    tpf = (eq_t & eq_p).astype(jnp.float32)
    C, RC = xt.shape
    g = RC // 128
    accp = ppf[:, 0:128]
    acct = tpf[:, 0:128]
    for j in range(1, g):
        accp = accp + ppf[:, j * 128:(j + 1) * 128]
        acct = acct + tpf[:, j * 128:(j + 1) * 128]
    ppa_ref[...] += accp
    tpa_ref[...] += acct

    @pl.when(i == nsteps - 1)
    def _fin():
        pp_out[...] = jnp.sum(ppa_ref[...], axis=1, keepdims=True)
        tp_out[...] = jnp.sum(tpa_ref[...], axis=1, keepdims=True)


def _fin_body(ytr_ref, ypr_ref, ppm_ref, tpm_ref, out_ref):
    xt = ytr_ref[...]
    xp = ypr_ref[...]
    mt = jnp.max(xt, axis=0, keepdims=True)
    mp = jnp.max(xp, axis=0, keepdims=True)
    eq_t = xt == mt
    eq_p = xp == mp
    pp = ppm_ref[...] + jnp.sum(eq_p.astype(jnp.float32), axis=1, keepdims=True)
    tp = tpm_ref[...] + jnp.sum((eq_t & eq_p).astype(jnp.float32), axis=1,
                                keepdims=True)
    C = pp.shape[0]
    prec = tp / (pp + _EPS)
    out_ref[...] = jnp.sum(prec, axis=0, keepdims=True) / jnp.float32(C)


def kernel(y_true, y_pred):
    N, C = y_true.shape
    ytT = y_true.T
    ypT = y_pred.T
    RC = 23808
    G = N // RC
    nmain = G * RC
    pp_m, tp_m = pl.pallas_call(
        functools.partial(_cls_body, G),
        grid=(G,),
        in_specs=[
            pl.BlockSpec((C, RC), lambda i: (0, i)),
            pl.BlockSpec((C, RC), lambda i: (0, i)),
        ],
        out_specs=[
            pl.BlockSpec((C, 1), lambda i: (0, 0)),
            pl.BlockSpec((C, 1), lambda i: (0, 0)),
        ],
        out_shape=[
            jax.ShapeDtypeStruct((C, 1), jnp.float32),
            jax.ShapeDtypeStruct((C, 1), jnp.float32),
        ],
        scratch_shapes=[
            pltpu.VMEM((C, 128), jnp.float32),
            pltpu.VMEM((C, 128), jnp.float32),
        ],
        compiler_params=pltpu.CompilerParams(dimension_semantics=("arbitrary",)),
    )(ytT, ypT)

    ytR = lax.slice(ytT, (0, nmain), (C, N))
    ypR = lax.slice(ypT, (0, nmain), (C, N))
    out = pl.pallas_call(
        _fin_body,
        out_shape=jax.ShapeDtypeStruct((1, 1), jnp.float32),
    )(ytR, ypR, pp_m, tp_m)
    return out[0, 0]


# speedup vs baseline: 5.9260x; 1.0007x over previous
import functools

import jax
import jax.numpy as jnp
from jax import lax
from jax.experimental import pallas as pl
from jax.experimental.pallas import tpu as pltpu

_EPS = float(jnp.finfo(jnp.float32).eps)


def _cls_body(nsteps, ytT_ref, ypT_ref, pp_out, tp_out, ppa_ref, tpa_ref):
    i = pl.program_id(0)

    @pl.when(i == 0)
    def _init():
        ppa_ref[...] = jnp.zeros_like(ppa_ref)
        tpa_ref[...] = jnp.zeros_like(tpa_ref)

    xt = ytT_ref[...]
    xp = ypT_ref[...]
    mt = jnp.max(xt, axis=0, keepdims=True)
    mp = jnp.max(xp, axis=0, keepdims=True)
    eq_t = xt == mt
    eq_p = xp == mp
    ppf = eq_p.astype(jnp.float32)
    tpf = (eq_t & eq_p).astype(jnp.float32)
    C, RC = xt.shape
    g = RC // 128
    accp = ppf[:, 0:128]
    acct = tpf[:, 0:128]
    for j in range(1, g):
        accp = accp + ppf[:, j * 128:(j + 1) * 128]
        acct = acct + tpf[:, j * 128:(j + 1) * 128]
    ppa_ref[...] += accp
    tpa_ref[...] += acct

    @pl.when(i == nsteps - 1)
    def _fin():
        pp_out[...] = jnp.sum(ppa_ref[...], axis=1, keepdims=True)
        tp_out[...] = jnp.sum(tpa_ref[...], axis=1, keepdims=True)


def _fin_body(ytr_ref, ypr_ref, ppm_ref, tpm_ref, out_ref):
    xt = ytr_ref[...]
    xp = ypr_ref[...]
    mt = jnp.max(xt, axis=0, keepdims=True)
    mp = jnp.max(xp, axis=0, keepdims=True)
    eq_t = xt == mt
    eq_p = xp == mp
    pp = ppm_ref[...] + jnp.sum(eq_p.astype(jnp.float32), axis=1, keepdims=True)
    tp = tpm_ref[...] + jnp.sum((eq_t & eq_p).astype(jnp.float32), axis=1,
                                keepdims=True)
    C = pp.shape[0]
    prec = tp / (pp + _EPS)
    out_ref[...] = jnp.sum(prec, axis=0, keepdims=True) / jnp.float32(C)


def kernel(y_true, y_pred):
    N, C = y_true.shape
    ytT = y_true.T
    ypT = y_pred.T
    RC = 27776
    G = N // RC
    nmain = G * RC
    pp_m, tp_m = pl.pallas_call(
        functools.partial(_cls_body, G),
        grid=(G,),
        in_specs=[
            pl.BlockSpec((C, RC), lambda i: (0, i)),
            pl.BlockSpec((C, RC), lambda i: (0, i)),
        ],
        out_specs=[
            pl.BlockSpec((C, 1), lambda i: (0, 0)),
            pl.BlockSpec((C, 1), lambda i: (0, 0)),
        ],
        out_shape=[
            jax.ShapeDtypeStruct((C, 1), jnp.float32),
            jax.ShapeDtypeStruct((C, 1), jnp.float32),
        ],
        scratch_shapes=[
            pltpu.VMEM((C, 128), jnp.float32),
            pltpu.VMEM((C, 128), jnp.float32),
        ],
        compiler_params=pltpu.CompilerParams(dimension_semantics=("arbitrary",), vmem_limit_bytes=120 * 1024 * 1024),
    )(ytT, ypT)

    ytR = lax.slice(ytT, (0, nmain), (C, N))
    ypR = lax.slice(ypT, (0, nmain), (C, N))
    out = pl.pallas_call(
        _fin_body,
        out_shape=jax.ShapeDtypeStruct((1, 1), jnp.float32),
    )(ytR, ypR, pp_m, tp_m)
    return out[0, 0]
